# baseline (speedup 1.0000x reference)
import functools

import jax
import jax.numpy as jnp
from jax import lax
from jax.experimental import pallas as pl
from jax.experimental.pallas import tpu as pltpu
from jax.experimental.pallas import tpu_sc as plsc

B = 256
S = 512
V = 100000
D = 768
EPS = 1e-12

NC = 2
NS = 16
L = 16
NW = NC * NS
SW = S // NW
CB = 2
T = CB * SW
G = B // CB
DC = D // L
N = B * S
NBUF = 4


def _body(seq_hbm, fseg_hbm, wtab_hbm, pos_hbm, seg_hbm, lnw_hbm, lnb_hbm,
          out_hbm, *scr):
    idx_s = scr[0:NBUF]
    fs_s = scr[NBUF:2 * NBUF]
    rows_s = scr[2 * NBUF:3 * NBUF]
    base_v, seg_v, delta_v, lnw_v, lnb_v = scr[3 * NBUF:3 * NBUF + 5]
    sem_s = scr[3 * NBUF + 5:4 * NBUF + 5]
    idxsem = scr[4 * NBUF + 5]

    wid = lax.axis_index("s") * NC + lax.axis_index("c")
    s0 = wid * SW

    pltpu.sync_copy(pos_hbm.at[pl.ds(s0, SW)], base_v)
    pltpu.sync_copy(seg_hbm, seg_v)
    pltpu.sync_copy(lnw_hbm, lnw_v)
    pltpu.sync_copy(lnb_hbm, lnb_v)

    def dinit(c, _):
        sl = pl.ds(c * L, L)
        delta_v[sl] = seg_v[1, sl] - seg_v[0, sl]
        return 0
    lax.fori_loop(0, DC, dinit, 0)

    def binit(i, _):
        r = i // DC
        sl = pl.ds((i % DC) * L, L)
        base_v[r, sl] = base_v[r, sl] + seg_v[0, sl]
        return 0
    lax.fori_loop(0, SW * DC, binit, 0)

    inv_d = jnp.float32(1.0 / D)

    def issue_idx(g, slot):
        hs = []
        for bl in range(CB):
            tok0 = (g * CB + bl) * S + s0
            hs.append(pltpu.async_copy(
                seq_hbm.at[pl.ds(tok0, SW)],
                idx_s[slot].at[pl.ds(bl * SW, SW)], idxsem))
            hs.append(pltpu.async_copy(
                fseg_hbm.at[pl.ds(tok0, SW)],
                fs_s[slot].at[pl.ds(bl * SW, SW)], idxsem))
        return hs

    def wait_idx(g, slot):
        for bl in range(CB):
            tok0 = (g * CB + bl) * S + s0
            pltpu.make_async_copy(
                seq_hbm.at[pl.ds(tok0, SW)],
                idx_s[slot].at[pl.ds(bl * SW, SW)], idxsem).wait()
            pltpu.make_async_copy(
                fseg_hbm.at[pl.ds(tok0, SW)],
                fs_s[slot].at[pl.ds(bl * SW, SW)], idxsem).wait()

    def issue_gather(slot):
        pltpu.async_copy(wtab_hbm.at[idx_s[slot]], rows_s[slot], sem_s[slot])

    def wait_gather(slot):
        pltpu.make_async_copy(
            wtab_hbm.at[idx_s[slot]], rows_s[slot], sem_s[slot]).wait()

    def issue_out(g, slot):
        for bl in range(CB):
            tok0 = (g * CB + bl) * S + s0
            pltpu.async_copy(rows_s[slot].at[pl.ds(bl * SW, SW)],
                             out_hbm.at[pl.ds(tok0, SW)], sem_s[slot])

    def wait_out(g, slot):
        for bl in range(CB):
            tok0 = (g * CB + bl) * S + s0
            pltpu.make_async_copy(rows_s[slot].at[pl.ds(bl * SW, SW)],
                                  out_hbm.at[pl.ds(tok0, SW)],
                                  sem_s[slot]).wait()

    def compute(slot):
        rows_v = rows_s[slot]
        fseg_v = fs_s[slot]

        def rstd(acc, acc2):
            m = jnp.broadcast_to(jnp.sum(acc), (L,)) * inv_d
            var = (jnp.broadcast_to(jnp.sum(acc2), (L,)) * inv_d
                   - m * m)
            x = jnp.maximum(var, 0.0) + EPS
            yi = jnp.int32(0x5F3759DF) - (
                lax.bitcast_convert_type(x, jnp.int32) >> 1)
            y = lax.bitcast_convert_type(yi, jnp.float32)
            for _ in range(3):
                y = y * (1.5 - 0.5 * x * y * y)
            return m, y

        TP = 8

        def grp(gi, _):
            SP = TP // CB
            sls = [gi * SP + k for k in range(SP)]
            ts = [bl * SW + gi * SP + k
                  for bl in range(CB) for k in range(SP)]
            fs = [plsc.load_gather(fseg_v, [jnp.full((L,), t, jnp.int32)])
                  for t in ts]
            z = jnp.zeros((L,), jnp.float32)

            @plsc.parallel_loop(0, DC, 1, unroll=4,
                                carry=(tuple([z] * TP), tuple([z] * TP)))
            def p1(c, carry):
                accs, accs2 = carry
                slc = pl.ds(c * L, L)
                d = delta_v[slc]
                bvs = [base_v[sl, slc] for sl in sls]
                na, nq = [], []
                for j in range(TP):
                    e = rows_v[ts[j], slc] + bvs[j % SP] + fs[j] * d
                    rows_v[ts[j], slc] = e
                    na.append(accs[j] + e)
                    nq.append(accs2[j] + e * e)
                return tuple(na), tuple(nq)

            accs, accs2 = p1
            mys = [rstd(accs[j], accs2[j]) for j in range(TP)]

            @plsc.parallel_loop(0, DC, 1, unroll=4)
            def p2(c):
                slc = pl.ds(c * L, L)
                w = lnw_v[slc]
                bb = lnb_v[slc]
                for j in range(TP):
                    m, y = mys[j]
                    rows_v[ts[j], slc] = ((rows_v[ts[j], slc] - m) * y * w
                                          + bb)

            return 0

        lax.fori_loop(0, T // TP, grp, 0)

    for g0 in range(2):
        hs = issue_idx(g0, g0)
        for h in hs:
            h.wait()
        issue_gather(g0)

    def outer(go, _):
        for p in range(NBUF):
            g = go * NBUF + p
            cur = p
            nxt = (p + 2) % NBUF
            wait_gather(cur)

            @pl.when(g + 2 < G)
            def _():
                issue_idx(g + 2, nxt)

            compute(cur)
            issue_out(g, cur)

            @pl.when(g + 2 < G)
            def _():
                @pl.when(g >= 2)
                def _():
                    wait_out(g - 2, nxt)
                wait_idx(g + 2, nxt)
                issue_gather(nxt)
        return 0

    lax.fori_loop(0, G // NBUF, outer, 0)

    for gg in range(G - 4, G):
        wait_out(gg, gg % NBUF)


@functools.partial(
    pl.kernel,
    out_type=jax.ShapeDtypeStruct((N, D), jnp.float32),
    mesh=plsc.VectorSubcoreMesh(core_axis_name="c", subcore_axis_name="s"),
    compiler_params=pltpu.CompilerParams(needs_layout_passes=False),
    scratch_types=(
        [pltpu.VMEM((T,), jnp.int32) for _ in range(NBUF)]
        + [pltpu.VMEM((T,), jnp.float32) for _ in range(NBUF)]
        + [pltpu.VMEM((T, D), jnp.float32) for _ in range(NBUF)]
        + [
            pltpu.VMEM((SW, D), jnp.float32),
            pltpu.VMEM((2, D), jnp.float32),
            pltpu.VMEM((D,), jnp.float32),
            pltpu.VMEM((D,), jnp.float32),
            pltpu.VMEM((D,), jnp.float32),
        ]
        + [pltpu.SemaphoreType.DMA for _ in range(NBUF)]
        + [pltpu.SemaphoreType.DMA]
    ),
)
def _emb_kernel(*refs):
    _body(*refs)


def kernel(seq, seq_seg, word_table, pos_table, seg_table, ln_w, ln_b):
    seq_flat = seq.reshape(N).astype(jnp.int32)
    fseg = seq_seg.reshape(N).astype(jnp.float32)
    out = _emb_kernel(seq_flat, fseg, word_table, pos_table, seg_table,
                      ln_w, ln_b)
    return out.reshape(B, S, D)


# speedup vs baseline: 1.2552x; 1.2552x over previous
import functools

import jax
import jax.numpy as jnp
from jax import lax
from jax.experimental import pallas as pl
from jax.experimental.pallas import tpu as pltpu
from jax.experimental.pallas import tpu_sc as plsc

B = 256
S = 512
V = 100000
D = 768
EPS = 1e-12

NC = 2
NS = 16
L = 16
NW = NC * NS
SW = S // NW
CB = 2
T = CB * SW
G = B // CB
DC = D // L
N = B * S
NBUF = 4


def _body(seq_hbm, fseg_hbm, wtab_hbm, pos_hbm, seg_hbm, lnw_hbm, lnb_hbm,
          out_hbm, *scr):
    idx_s = scr[0:NBUF]
    fs_s = scr[NBUF:2 * NBUF]
    rows_s = scr[2 * NBUF:3 * NBUF]
    base_v, seg_v, delta_v, lnw_v, lnb_v = scr[3 * NBUF:3 * NBUF + 5]
    sem_s = scr[3 * NBUF + 5:4 * NBUF + 5]
    idxsem = scr[4 * NBUF + 5]

    wid = lax.axis_index("s") * NC + lax.axis_index("c")
    s0 = wid * SW

    pltpu.sync_copy(pos_hbm.at[pl.ds(s0, SW)], base_v)
    pltpu.sync_copy(seg_hbm, seg_v)
    pltpu.sync_copy(lnw_hbm, lnw_v)
    pltpu.sync_copy(lnb_hbm, lnb_v)

    def dinit(c, _):
        sl = pl.ds(c * L, L)
        delta_v[sl] = seg_v[1, sl] - seg_v[0, sl]
        return 0
    lax.fori_loop(0, DC, dinit, 0)

    def binit(i, _):
        r = i // DC
        sl = pl.ds((i % DC) * L, L)
        base_v[r, sl] = base_v[r, sl] + seg_v[0, sl]
        return 0
    lax.fori_loop(0, SW * DC, binit, 0)

    inv_d = jnp.float32(1.0 / D)

    def issue_idx(g, slot):
        hs = []
        for bl in range(CB):
            tok0 = (g * CB + bl) * S + s0
            hs.append(pltpu.async_copy(
                seq_hbm.at[pl.ds(tok0, SW)],
                idx_s[slot].at[pl.ds(bl * SW, SW)], idxsem))
            hs.append(pltpu.async_copy(
                fseg_hbm.at[pl.ds(tok0, SW)],
                fs_s[slot].at[pl.ds(bl * SW, SW)], idxsem))
        return hs

    def wait_idx(g, slot):
        for bl in range(CB):
            tok0 = (g * CB + bl) * S + s0
            pltpu.make_async_copy(
                seq_hbm.at[pl.ds(tok0, SW)],
                idx_s[slot].at[pl.ds(bl * SW, SW)], idxsem).wait()
            pltpu.make_async_copy(
                fseg_hbm.at[pl.ds(tok0, SW)],
                fs_s[slot].at[pl.ds(bl * SW, SW)], idxsem).wait()

    def issue_gather(slot):
        pltpu.async_copy(wtab_hbm.at[idx_s[slot]], rows_s[slot], sem_s[slot])

    def wait_gather(slot):
        pltpu.make_async_copy(
            wtab_hbm.at[idx_s[slot]], rows_s[slot], sem_s[slot]).wait()

    def issue_out(g, slot):
        for bl in range(CB):
            tok0 = (g * CB + bl) * S + s0
            pltpu.async_copy(rows_s[slot].at[pl.ds(bl * SW, SW)],
                             out_hbm.at[pl.ds(tok0, SW)], sem_s[slot])

    def wait_out(g, slot):
        for bl in range(CB):
            tok0 = (g * CB + bl) * S + s0
            pltpu.make_async_copy(rows_s[slot].at[pl.ds(bl * SW, SW)],
                                  out_hbm.at[pl.ds(tok0, SW)],
                                  sem_s[slot]).wait()

    def compute(slot):
        rows_v = rows_s[slot]
        fseg_v = fs_s[slot]

        def rstd(acc, acc2):
            m = jnp.broadcast_to(jnp.sum(acc), (L,)) * inv_d
            var = (jnp.broadcast_to(jnp.sum(acc2), (L,)) * inv_d
                   - m * m)
            x = jnp.maximum(var, 0.0) + EPS
            yi = jnp.int32(0x5F3759DF) - (
                lax.bitcast_convert_type(x, jnp.int32) >> 1)
            y = lax.bitcast_convert_type(yi, jnp.float32)
            for _ in range(3):
                y = y * (1.5 - 0.5 * x * y * y)
            return m, y

        TP = 8

        def grp(gi, _):
            ts = [gi * TP + j for j in range(TP)]
            sls = [t % SW for t in ts]
            fs = [plsc.load_gather(fseg_v, [jnp.full((L,), t, jnp.int32)])
                  for t in ts]
            z = jnp.zeros((L,), jnp.float32)

            @plsc.parallel_loop(0, DC, 1, unroll=4,
                                carry=(tuple([z] * TP), tuple([z] * TP)))
            def p1(c, carry):
                accs, accs2 = carry
                slc = pl.ds(c * L, L)
                d = delta_v[slc]
                na, nq = [], []
                for j in range(TP):
                    e = rows_v[ts[j], slc] + base_v[sls[j], slc] + fs[j] * d
                    rows_v[ts[j], slc] = e
                    na.append(accs[j] + e)
                    nq.append(accs2[j] + e * e)
                return tuple(na), tuple(nq)

            accs, accs2 = p1
            mys = [rstd(accs[j], accs2[j]) for j in range(TP)]

            @plsc.parallel_loop(0, DC, 1, unroll=4)
            def p2(c):
                slc = pl.ds(c * L, L)
                w = lnw_v[slc]
                bb = lnb_v[slc]
                for j in range(TP):
                    m, y = mys[j]
                    rows_v[ts[j], slc] = ((rows_v[ts[j], slc] - m) * y * w
                                          + bb)

            return 0

        lax.fori_loop(0, T // TP, grp, 0)

    for g0 in range(2):
        hs = issue_idx(g0, g0)
        for h in hs:
            h.wait()
        issue_gather(g0)

    def outer(go, _):
        for p in range(NBUF):
            g = go * NBUF + p
            cur = p
            nxt = (p + 2) % NBUF
            wait_gather(cur)

            @pl.when(g + 2 < G)
            def _():
                issue_idx(g + 2, nxt)

            compute(cur)
            issue_out(g, cur)

            @pl.when(g + 2 < G)
            def _():
                @pl.when(g >= 2)
                def _():
                    wait_out(g - 2, nxt)
                wait_idx(g + 2, nxt)
                issue_gather(nxt)
        return 0

    lax.fori_loop(0, G // NBUF, outer, 0)

    for gg in range(G - 4, G):
        wait_out(gg, gg % NBUF)


@functools.partial(
    pl.kernel,
    out_type=jax.ShapeDtypeStruct((N, D), jnp.float32),
    mesh=plsc.VectorSubcoreMesh(core_axis_name="c", subcore_axis_name="s"),
    compiler_params=pltpu.CompilerParams(needs_layout_passes=False),
    scratch_types=(
        [pltpu.VMEM((T,), jnp.int32) for _ in range(NBUF)]
        + [pltpu.VMEM((T,), jnp.float32) for _ in range(NBUF)]
        + [pltpu.VMEM((T, D), jnp.float32) for _ in range(NBUF)]
        + [
            pltpu.VMEM((SW, D), jnp.float32),
            pltpu.VMEM((2, D), jnp.float32),
            pltpu.VMEM((D,), jnp.float32),
            pltpu.VMEM((D,), jnp.float32),
            pltpu.VMEM((D,), jnp.float32),
        ]
        + [pltpu.SemaphoreType.DMA for _ in range(NBUF)]
        + [pltpu.SemaphoreType.DMA]
    ),
)
def _emb_kernel(*refs):
    _body(*refs)


def kernel(seq, seq_seg, word_table, pos_table, seg_table, ln_w, ln_b):
    seq_flat = seq.reshape(N).astype(jnp.int32)
    fseg = seq_seg.reshape(N).astype(jnp.float32)
    out = _emb_kernel(seq_flat, fseg, word_table, pos_table, seg_table,
                      ln_w, ln_b)
    return out.reshape(B, S, D)
